# baseline (speedup 1.0000x reference)
import jax
import jax.numpy as jnp
from jax.experimental import pallas as pl
from jax.experimental.pallas import tpu as pltpu

B = 1024
DATA_DIM = 128
COORD = 2
V = DATA_DIM // COORD
HIDDEN = 256
TDIM = 128
N = B * V

BB = 128
R = BB * V


def _silu(v):
    return 0.5 * v * (1.0 + jnp.tanh(0.5 * v))


def _spmm_vpu(u3):
    nxt = jnp.concatenate([u3[:, 1:], u3[:, :1]], axis=1)
    prv = jnp.concatenate([u3[:, -1:], u3[:, :-1]], axis=1)
    return (u3 + nxt + prv) * jnp.float32(1.0 / 3.0)


def _spmm(u3, A3):
    Ab = jnp.broadcast_to(A3[None], (BB, V, V))
    return jax.lax.dot_general(Ab, u3, (((2,), (1,)), ((0,), (0,))),
                               preferred_element_type=jnp.float32)


def _body(coords, tcol, Wt, bt, W0, b0, W1, b1, W2, b2, W3, b3, Wres,
          Wh1, bh1, Wh2, bh2, out_ref):
    f32 = jnp.float32
    dot = lambda a, b: jnp.dot(a, b, preferred_element_type=f32)

    ri = jax.lax.broadcasted_iota(jnp.int32, (V, V), 0)
    ci = jax.lax.broadcasted_iota(jnp.int32, (V, V), 1)
    dd = jnp.abs(ri - ci)
    A3 = jnp.where((dd == 0) | (dd == 1) | (dd == V - 1),
                   f32(1.0 / 3.0), f32(0.0))

    li = jax.lax.broadcasted_iota(jnp.int32, (1, TDIM), 1)
    lm = jnp.where(li >= TDIM // 2, li - TDIM // 2, li).astype(f32)
    freqs = jnp.exp(f32(-jnp.log(10000.0) / (TDIM // 2 - 1)) * lm)
    phase = jnp.where(li >= TDIM // 2, f32(jnp.pi / 2), f32(0.0))
    tf = tcol[...].astype(f32) * freqs + phase

    te = _silu(dot(jnp.sin(tf), Wt[...]) + bt[...])
    Wtp = W0[COORD:, :] + Wres[COORD:, :]
    c0 = dot(te, Wtp) + b0[...]

    c = coords[...]
    sc = _spmm(c.reshape(BB, V, COORD), A3).reshape(R, COORD)
    pre = dot(sc, W0[:COORD, :]) + dot(c, Wres[:COORD, :])
    h = _silu(pre.reshape(BB, V, HIDDEN) + c0[:, None, :]).reshape(R, HIDDEN)

    bf = jnp.bfloat16
    for W, b in ((W1, b1), (W2, b2), (W3, b3)):
        u = dot(h.astype(bf), W[...].astype(bf))
        s = _spmm(u.reshape(BB, V, HIDDEN), A3).reshape(R, HIDDEN)
        h = _silu(s + b[...] + h)

    g = _silu(dot(h.astype(bf), Wh1[...].astype(bf)) + bh1[...])
    out_ref[...] = dot(g, Wh2[...]) + bh2[...]


@jax.jit
def kernel(x, t, Wt, bt, W0, b0, W1, b1, W2, b2, W3, b3, Wres,
           Wh1, bh1, Wh2, bh2):
    grid = B // BB
    rep = lambda i: (0, 0)
    row = lambda v: v.reshape(1, -1)

    out = pl.pallas_call(
        _body,
        grid=(grid,),
        in_specs=[
            pl.BlockSpec((R, COORD), lambda i: (i, 0)),
            pl.BlockSpec((BB, 1), lambda i: (i, 0)),
            pl.BlockSpec((TDIM, TDIM), rep),
            pl.BlockSpec((1, TDIM), rep),
            pl.BlockSpec((COORD + TDIM, HIDDEN), rep),
            pl.BlockSpec((1, HIDDEN), rep),
            pl.BlockSpec((HIDDEN, HIDDEN), rep),
            pl.BlockSpec((1, HIDDEN), rep),
            pl.BlockSpec((HIDDEN, HIDDEN), rep),
            pl.BlockSpec((1, HIDDEN), rep),
            pl.BlockSpec((HIDDEN, HIDDEN), rep),
            pl.BlockSpec((1, HIDDEN), rep),
            pl.BlockSpec((COORD + TDIM, HIDDEN), rep),
            pl.BlockSpec((HIDDEN, HIDDEN), rep),
            pl.BlockSpec((1, HIDDEN), rep),
            pl.BlockSpec((HIDDEN, COORD), rep),
            pl.BlockSpec((1, COORD), rep),
        ],
        out_specs=pl.BlockSpec((R, COORD), lambda i: (i, 0)),
        out_shape=jax.ShapeDtypeStruct((N, COORD), jnp.float32),
        compiler_params=pltpu.CompilerParams(
            dimension_semantics=("parallel",)),
    )(x.reshape(N, COORD), t.reshape(B, 1), Wt, row(bt), W0, row(b0),
      W1, row(b1), W2, row(b2), W3, row(b3), Wres,
      Wh1, row(bh1), Wh2, row(bh2))

    return out.reshape(B, DATA_DIM)


# speedup vs baseline: 1.4078x; 1.4078x over previous
import jax
import jax.numpy as jnp
from jax.experimental import pallas as pl
from jax.experimental.pallas import tpu as pltpu

B = 1024
DATA_DIM = 128
COORD = 2
V = DATA_DIM // COORD
HIDDEN = 256
TDIM = 128
N = B * V

BB = 128
R = BB * V


def _silu(v):
    return 0.5 * v * (1.0 + jnp.tanh(0.5 * v))


def _spmm_vpu(u3):
    nxt = jnp.concatenate([u3[:, 1:], u3[:, :1]], axis=1)
    prv = jnp.concatenate([u3[:, -1:], u3[:, :-1]], axis=1)
    return (u3 + nxt + prv) * jnp.float32(1.0 / 3.0)


def _spmm(u3, A3):
    Ab = jnp.broadcast_to(A3[None], (BB, V, V))
    return jax.lax.dot_general(Ab, u3, (((2,), (1,)), ((0,), (0,))),
                               preferred_element_type=jnp.float32)


def _body(coords, tcol, Wt, bt, W0, b0, W1, b1, W2, b2, W3, b3, Wres,
          Wh1, bh1, Wh2, bh2, out_ref):
    f32 = jnp.float32
    dot = lambda a, b: jnp.dot(a, b, preferred_element_type=f32)

    ri = jax.lax.broadcasted_iota(jnp.int32, (V, V), 0)
    ci = jax.lax.broadcasted_iota(jnp.int32, (V, V), 1)
    dd = jnp.abs(ri - ci)
    A3 = jnp.where((dd == 0) | (dd == 1) | (dd == V - 1),
                   f32(1.0 / 3.0), f32(0.0))

    li = jax.lax.broadcasted_iota(jnp.int32, (1, TDIM), 1)
    lm = jnp.where(li >= TDIM // 2, li - TDIM // 2, li).astype(f32)
    freqs = jnp.exp(f32(-jnp.log(10000.0) / (TDIM // 2 - 1)) * lm)
    phase = jnp.where(li >= TDIM // 2, f32(jnp.pi / 2), f32(0.0))
    tf = tcol[...].astype(f32) * freqs + phase

    te = _silu(dot(jnp.sin(tf), Wt[...]) + bt[...])
    Wtp = W0[COORD:, :] + Wres[COORD:, :]
    c0 = dot(te, Wtp) + b0[...]

    ri = jax.lax.broadcasted_iota(jnp.int32, (4 * V, DATA_DIM), 0)
    li2 = jax.lax.broadcasted_iota(jnp.int32, (4 * V, DATA_DIM), 1)
    vv = ri & (V - 1)
    kk = ri >> 6
    du = jnp.abs((li2 >> 1) - vv)
    lane_c = (li2 & 1) == (kk & 1)
    near = (du == 0) | (du == 1) | (du == V - 1)
    Lsel = jnp.where(lane_c & (kk >= 2) & near, f32(1.0 / 3.0),
                     jnp.where(lane_c & (kk < 2) & (du == 0), f32(1.0),
                               f32(0.0)))
    x3 = coords[...].reshape(BB, DATA_DIM, 1)
    P = jax.lax.dot_general(jnp.broadcast_to(Lsel[None], (BB, 4 * V, DATA_DIM)),
                            x3, (((2,), (1,)), ((0,), (0,))),
                            preferred_element_type=f32)
    Xa = jnp.concatenate([P[:, 0:V], P[:, V:2 * V],
                          P[:, 2 * V:3 * V], P[:, 3 * V:4 * V]], axis=2)
    Wcomb = jnp.concatenate([Wres[:COORD, :], W0[:COORD, :]], axis=0)
    pre = jax.lax.dot_general(Xa, jnp.broadcast_to(Wcomb[None],
                                                   (BB, 4, HIDDEN)),
                              (((2,), (1,)), ((0,), (0,))),
                              preferred_element_type=f32)
    h = _silu(pre + c0[:, None, :]).reshape(R, HIDDEN)

    for W, b in ((W1, b1), (W2, b2), (W3, b3)):
        u = dot(h, W[...])
        s = _spmm(u.reshape(BB, V, HIDDEN), A3).reshape(R, HIDDEN)
        h = _silu(s + b[...] + h)

    g = _silu(dot(h, Wh1[...]) + bh1[...])
    res3 = (dot(g, Wh2[...]) + bh2[...]).reshape(BB, V, COORD)
    vi = jax.lax.broadcasted_iota(jnp.int32, (V, DATA_DIM), 0)
    lo = jax.lax.broadcasted_iota(jnp.int32, (V, DATA_DIM), 1)
    U = jnp.where(lo == 2 * vi, f32(1.0), f32(0.0))
    r2 = jax.lax.dot_general(res3, jnp.broadcast_to(U[None],
                                                    (BB, V, DATA_DIM)),
                             (((1,), (1,)), ((0,), (0,))),
                             preferred_element_type=f32)
    lm1 = jax.lax.broadcasted_iota(jnp.int32, (1, DATA_DIM), 1)
    odd = pltpu.roll(r2[:, 1, :], 1, 1)
    out_ref[...] = jnp.where((lm1 & 1) == 0, r2[:, 0, :], odd)


@jax.jit
def kernel(x, t, Wt, bt, W0, b0, W1, b1, W2, b2, W3, b3, Wres,
           Wh1, bh1, Wh2, bh2):
    grid = B // BB
    rep = lambda i: (0, 0)
    row = lambda v: v.reshape(1, -1)

    out = pl.pallas_call(
        _body,
        grid=(grid,),
        in_specs=[
            pl.BlockSpec((BB, DATA_DIM), lambda i: (i, 0)),
            pl.BlockSpec((BB, 1), lambda i: (i, 0)),
            pl.BlockSpec((TDIM, TDIM), rep),
            pl.BlockSpec((1, TDIM), rep),
            pl.BlockSpec((COORD + TDIM, HIDDEN), rep),
            pl.BlockSpec((1, HIDDEN), rep),
            pl.BlockSpec((HIDDEN, HIDDEN), rep),
            pl.BlockSpec((1, HIDDEN), rep),
            pl.BlockSpec((HIDDEN, HIDDEN), rep),
            pl.BlockSpec((1, HIDDEN), rep),
            pl.BlockSpec((HIDDEN, HIDDEN), rep),
            pl.BlockSpec((1, HIDDEN), rep),
            pl.BlockSpec((COORD + TDIM, HIDDEN), rep),
            pl.BlockSpec((HIDDEN, HIDDEN), rep),
            pl.BlockSpec((1, HIDDEN), rep),
            pl.BlockSpec((HIDDEN, COORD), rep),
            pl.BlockSpec((1, COORD), rep),
        ],
        out_specs=pl.BlockSpec((BB, DATA_DIM), lambda i: (i, 0)),
        out_shape=jax.ShapeDtypeStruct((B, DATA_DIM), jnp.float32),
        compiler_params=pltpu.CompilerParams(
            dimension_semantics=("parallel",)),
    )(x, t.reshape(B, 1), Wt, row(bt), W0, row(b0),
      W1, row(b1), W2, row(b2), W3, row(b3), Wres,
      Wh1, row(bh1), Wh2, row(bh2))

    return out


# speedup vs baseline: 1.6553x; 1.1758x over previous
import jax
import jax.numpy as jnp
from jax.experimental import pallas as pl
from jax.experimental.pallas import tpu as pltpu

B = 1024
DATA_DIM = 128
COORD = 2
V = DATA_DIM // COORD
HIDDEN = 256
TDIM = 128
N = B * V

BB = 128
R = BB * V


def _silu(v):
    return 0.5 * v * (1.0 + jnp.tanh(0.5 * v))


def _spmm_rows(u):
    nxt = jnp.concatenate([u[BB:], u[:BB]], axis=0)
    prv = jnp.concatenate([u[-BB:], u[:-BB]], axis=0)
    return (u + nxt + prv) * jnp.float32(1.0 / 3.0)


def _body(coords, tcol, Wt, bt, W0, b0, W1, b1, W2, b2, W3, b3, Wres,
          Wh1, bh1, Wh2, bh2, out_ref, scr):
    f32 = jnp.float32
    dot = lambda a, b: jnp.dot(a, b, preferred_element_type=f32)

    li = jax.lax.broadcasted_iota(jnp.int32, (1, TDIM), 1)
    lm = jnp.where(li >= TDIM // 2, li - TDIM // 2, li).astype(f32)
    freqs = jnp.exp(f32(-jnp.log(10000.0) / (TDIM // 2 - 1)) * lm)
    phase = jnp.where(li >= TDIM // 2, f32(jnp.pi / 2), f32(0.0))
    tf = tcol[...].astype(f32) * freqs + phase

    te = _silu(dot(jnp.sin(tf), Wt[...]) + bt[...])
    Wtp = W0[COORD:, :] + Wres[COORD:, :]
    c0 = dot(te, Wtp) + b0[...]
    c0t = jnp.broadcast_to(c0[None], (V, BB, HIDDEN)).reshape(R, HIDDEN)

    xT = jnp.swapaxes(coords[...], 0, 1)
    xv = xT.reshape(V, COORD, BB)
    c2 = jnp.swapaxes(xv, 1, 2).reshape(R, COORD)

    pre = dot(_spmm_rows(c2), W0[:COORD, :]) + dot(c2, Wres[:COORD, :])
    h = _silu(pre + c0t)

    third = f32(1.0 / 3.0)
    for W, b in ((W1, b1), (W2, b2), (W3, b3)):
        u = dot(h, W[...] * third)
        scr[BB:BB + R, :] = u
        scr[0:BB, :] = u[R - BB:R, :]
        scr[BB + R:, :] = u[0:BB, :]
        s = scr[0:R, :] + scr[BB:BB + R, :] + scr[2 * BB:, :]
        h = _silu(s + b[...] + h)

    g = _silu(dot(h, Wh1[...]) + bh1[...])
    res = dot(g, Wh2[...]) + bh2[...]
    rv = jnp.swapaxes(res.reshape(V, BB, COORD), 1, 2)
    out_ref[...] = jnp.swapaxes(rv.reshape(DATA_DIM, BB), 0, 1)


@jax.jit
def kernel(x, t, Wt, bt, W0, b0, W1, b1, W2, b2, W3, b3, Wres,
           Wh1, bh1, Wh2, bh2):
    grid = B // BB
    rep = lambda i: (0, 0)
    row = lambda v: v.reshape(1, -1)

    out = pl.pallas_call(
        _body,
        grid=(grid,),
        in_specs=[
            pl.BlockSpec((BB, DATA_DIM), lambda i: (i, 0)),
            pl.BlockSpec((BB, 1), lambda i: (i, 0)),
            pl.BlockSpec((TDIM, TDIM), rep),
            pl.BlockSpec((1, TDIM), rep),
            pl.BlockSpec((COORD + TDIM, HIDDEN), rep),
            pl.BlockSpec((1, HIDDEN), rep),
            pl.BlockSpec((HIDDEN, HIDDEN), rep),
            pl.BlockSpec((1, HIDDEN), rep),
            pl.BlockSpec((HIDDEN, HIDDEN), rep),
            pl.BlockSpec((1, HIDDEN), rep),
            pl.BlockSpec((HIDDEN, HIDDEN), rep),
            pl.BlockSpec((1, HIDDEN), rep),
            pl.BlockSpec((COORD + TDIM, HIDDEN), rep),
            pl.BlockSpec((HIDDEN, HIDDEN), rep),
            pl.BlockSpec((1, HIDDEN), rep),
            pl.BlockSpec((HIDDEN, COORD), rep),
            pl.BlockSpec((1, COORD), rep),
        ],
        out_specs=pl.BlockSpec((BB, DATA_DIM), lambda i: (i, 0)),
        out_shape=jax.ShapeDtypeStruct((B, DATA_DIM), jnp.float32),
        scratch_shapes=[pltpu.VMEM((R + 2 * BB, HIDDEN), jnp.float32)],
        compiler_params=pltpu.CompilerParams(
            dimension_semantics=("parallel",)),
    )(x, t.reshape(B, 1), Wt, row(bt), W0, row(b0),
      W1, row(b1), W2, row(b2), W3, row(b3), Wres,
      Wh1, row(bh1), Wh2, row(bh2))

    return out


# speedup vs baseline: 1.6599x; 1.0027x over previous
import jax
import jax.numpy as jnp
from jax.experimental import pallas as pl
from jax.experimental.pallas import tpu as pltpu

B = 1024
DATA_DIM = 128
COORD = 2
V = DATA_DIM // COORD
HIDDEN = 256
TDIM = 128
N = B * V

BB = 128
R = BB * V


def _silu(v):
    return 0.5 * v * (1.0 + jnp.tanh(0.5 * v))


def _spmm_rows(u):
    nxt = jnp.concatenate([u[BB:], u[:BB]], axis=0)
    prv = jnp.concatenate([u[-BB:], u[:-BB]], axis=0)
    return (u + nxt + prv) * jnp.float32(1.0 / 3.0)


def _body(coords, tcol, Wt, bt, W0, b0, W1, b1, W2, b2, W3, b3, Wres,
          Wh1, bh1, Wh2, bh2, out_ref):
    f32 = jnp.float32
    dot = lambda a, b: jnp.dot(a, b, preferred_element_type=f32)

    li = jax.lax.broadcasted_iota(jnp.int32, (1, TDIM), 1)
    lm = jnp.where(li >= TDIM // 2, li - TDIM // 2, li).astype(f32)
    freqs = jnp.exp(f32(-jnp.log(10000.0) / (TDIM // 2 - 1)) * lm)
    phase = jnp.where(li >= TDIM // 2, f32(jnp.pi / 2), f32(0.0))
    tf = tcol[...].astype(f32) * freqs + phase

    te = _silu(dot(jnp.sin(tf), Wt[...]) + bt[...])
    Wtp = W0[COORD:, :] + Wres[COORD:, :]
    c0 = dot(te, Wtp) + b0[...]
    c0t = jnp.broadcast_to(c0[None], (V, BB, HIDDEN)).reshape(R, HIDDEN)

    xT = jnp.swapaxes(coords[...], 0, 1)
    xv = xT.reshape(V, COORD, BB)
    c2 = jnp.swapaxes(xv, 1, 2).reshape(R, COORD)

    pre = dot(_spmm_rows(c2), W0[:COORD, :]) + dot(c2, Wres[:COORD, :])
    h = _silu(pre + c0t)

    third = f32(1.0 / 3.0)
    for W, b in ((W1, b1), (W2, b2), (W3, b3)):
        u = dot(h, W[...] * third)
        bb = b[...]
        top = _silu(u[R - BB:] + u[:BB] + u[BB:2 * BB] + bb + h[:BB])
        mid = _silu(u[:R - 2 * BB] + u[BB:R - BB] + u[2 * BB:] + bb
                    + h[BB:R - BB])
        bot = _silu(u[R - 2 * BB:R - BB] + u[R - BB:] + u[:BB] + bb
                    + h[R - BB:])
        h = jnp.concatenate([top, mid, bot], axis=0)

    g = _silu(dot(h, Wh1[...]) + bh1[...])
    res = dot(g, Wh2[...]) + bh2[...]
    rv = jnp.swapaxes(res.reshape(V, BB, COORD), 1, 2)
    out_ref[...] = jnp.swapaxes(rv.reshape(DATA_DIM, BB), 0, 1)


@jax.jit
def kernel(x, t, Wt, bt, W0, b0, W1, b1, W2, b2, W3, b3, Wres,
           Wh1, bh1, Wh2, bh2):
    grid = B // BB
    rep = lambda i: (0, 0)
    row = lambda v: v.reshape(1, -1)

    out = pl.pallas_call(
        _body,
        grid=(grid,),
        in_specs=[
            pl.BlockSpec((BB, DATA_DIM), lambda i: (i, 0)),
            pl.BlockSpec((BB, 1), lambda i: (i, 0)),
            pl.BlockSpec((TDIM, TDIM), rep),
            pl.BlockSpec((1, TDIM), rep),
            pl.BlockSpec((COORD + TDIM, HIDDEN), rep),
            pl.BlockSpec((1, HIDDEN), rep),
            pl.BlockSpec((HIDDEN, HIDDEN), rep),
            pl.BlockSpec((1, HIDDEN), rep),
            pl.BlockSpec((HIDDEN, HIDDEN), rep),
            pl.BlockSpec((1, HIDDEN), rep),
            pl.BlockSpec((HIDDEN, HIDDEN), rep),
            pl.BlockSpec((1, HIDDEN), rep),
            pl.BlockSpec((COORD + TDIM, HIDDEN), rep),
            pl.BlockSpec((HIDDEN, HIDDEN), rep),
            pl.BlockSpec((1, HIDDEN), rep),
            pl.BlockSpec((HIDDEN, COORD), rep),
            pl.BlockSpec((1, COORD), rep),
        ],
        out_specs=pl.BlockSpec((BB, DATA_DIM), lambda i: (i, 0)),
        out_shape=jax.ShapeDtypeStruct((B, DATA_DIM), jnp.float32),
        compiler_params=pltpu.CompilerParams(
            dimension_semantics=("parallel",)),
    )(x, t.reshape(B, 1), Wt, row(bt), W0, row(b0),
      W1, row(b1), W2, row(b2), W3, row(b3), Wres,
      Wh1, row(bh1), Wh2, row(bh2))

    return out
